# baseline (speedup 1.0000x reference)
import functools

import jax
import jax.numpy as jnp
from jax import lax
from jax.experimental import pallas as pl
from jax.experimental.pallas import tpu as pltpu
from jax.experimental.pallas import tpu_sc as plsc

BATCH = 4096
NEG = 200
DIM = 64
NC = 2
NS = 16
NW = NC * NS
BPW = BATCH // NW
LANES = 16
NEG_A = 104
NEG_B = NEG - NEG_A
NEG_PAD = 208


def _score_body(hidx_hbm, ridx_hbm, pidx_hbm, tidx_hbm, ent_hbm, rel_hbm,
                posi_out, nega_out,
                hidx_v, ridx_v, pidx_v, tidx_v,
                hrows, rrows, prows, hr_rows, posi_buf,
                trows0, trows1, sbuf0, sbuf1,
                psem, gsem0, gsem1, ssem0, ssem1):
    wid = lax.axis_index("s") * NC + lax.axis_index("c")
    base = wid * BPW

    pltpu.sync_copy(hidx_hbm.at[pl.ds(base, BPW)], hidx_v)
    pltpu.sync_copy(ridx_hbm.at[pl.ds(base, BPW)], ridx_v)
    pltpu.sync_copy(pidx_hbm.at[pl.ds(base, BPW)], pidx_v)
    pltpu.sync_copy(tidx_hbm.at[pl.ds(base * NEG, BPW * NEG)], tidx_v)

    c_h = pltpu.async_copy(ent_hbm.at[hidx_v], hrows, psem)
    c_r = pltpu.async_copy(rel_hbm.at[ridx_v], rrows, psem)
    c_p = pltpu.async_copy(ent_hbm.at[pidx_v], prows, psem)

    def issue_tail_gather(b, trows, gsem):
        off = b * NEG
        pltpu.async_copy(ent_hbm.at[tidx_v.at[pl.ds(off, NEG_A)]],
                         trows.at[pl.ds(0, NEG_A)], gsem)
        pltpu.async_copy(ent_hbm.at[tidx_v.at[pl.ds(off + NEG_A, NEG_B)]],
                         trows.at[pl.ds(NEG_A, NEG_B)], gsem)

    def drain_tail_gather(trows, gsem):
        pltpu.make_async_copy(ent_hbm.at[pl.ds(0, NEG)],
                              trows.at[pl.ds(0, NEG)], gsem).wait()

    issue_tail_gather(0, trows0, gsem0)
    issue_tail_gather(1, trows1, gsem1)

    c_h.wait()
    c_r.wait()
    c_p.wait()

    lane_iota = lax.iota(jnp.int32, LANES)
    NGRP = NEG_PAD // LANES

    def hr_body(b, carry):
        for c in range(DIM // LANES):
            h = hrows[b, pl.ds(c * LANES, LANES)]
            r = rrows[b, pl.ds(c * LANES, LANES)]
            hr_rows[b, pl.ds(c * LANES, LANES)] = h * r
        return carry

    lax.fori_loop(0, BPW, hr_body, 0, unroll=False)

    def posi_group(g, carry):
        rows = lane_iota + g * LANES

        def d_body(d, acc):
            col = jnp.full((LANES,), d, jnp.int32)
            gh = plsc.load_gather(hr_rows, [rows, col])
            gp = plsc.load_gather(prows, [rows, col])
            return acc + gh * gp

        acc = lax.fori_loop(0, DIM, d_body,
                            jnp.zeros((LANES,), jnp.float32), unroll=False)
        posi_buf[pl.ds(g * LANES, LANES)] = acc
        return carry

    lax.fori_loop(0, BPW // LANES, posi_group, 0, unroll=False)

    def compute_scores(b, trows, sbuf):
        brow = jnp.full((LANES,), b, jnp.int32)

        def d_body(d, accs):
            col = jnp.full((LANES,), d, jnp.int32)
            hv = plsc.load_gather(hr_rows, [brow, col])
            return tuple(
                accs[j] + hv * plsc.load_gather(
                    trows, [lane_iota + j * LANES, col])
                for j in range(NGRP))

        accs = lax.fori_loop(
            0, DIM, d_body,
            tuple(jnp.zeros((LANES,), jnp.float32) for _ in range(NGRP)),
            unroll=False)
        for j in range(NGRP):
            sbuf[pl.ds(j * LANES, LANES)] = accs[j]

    def half(i, b, trows, gsem, sbuf, ssem):
        drain_tail_gather(trows, gsem)

        @pl.when(i > 0)
        def _():
            pltpu.make_async_copy(sbuf.at[pl.ds(0, NEG)],
                                  nega_out.at[pl.ds(0, NEG)], ssem).wait()

        compute_scores(b, trows, sbuf)
        pltpu.async_copy(sbuf.at[pl.ds(0, NEG)],
                         nega_out.at[pl.ds((base + b) * NEG, NEG)], ssem)

        @pl.when(b + 2 < BPW)
        def _():
            issue_tail_gather(b + 2, trows, gsem)

    def outer(i, carry):
        half(i, 2 * i, trows0, gsem0, sbuf0, ssem0)
        half(i, 2 * i + 1, trows1, gsem1, sbuf1, ssem1)
        return carry

    lax.fori_loop(0, BPW // 2, outer, 0, unroll=False)

    pltpu.make_async_copy(sbuf0.at[pl.ds(0, NEG)],
                          nega_out.at[pl.ds(0, NEG)], ssem0).wait()
    pltpu.make_async_copy(sbuf1.at[pl.ds(0, NEG)],
                          nega_out.at[pl.ds(0, NEG)], ssem1).wait()
    pltpu.sync_copy(posi_buf, posi_out.at[pl.ds(base, BPW)])


@functools.cache
def _sc_score_kernel():
    return pl.kernel(
        _score_body,
        out_type=[
            jax.ShapeDtypeStruct((BATCH,), jnp.float32),
            jax.ShapeDtypeStruct((BATCH * NEG,), jnp.float32),
        ],
        mesh=plsc.VectorSubcoreMesh(core_axis_name="c", subcore_axis_name="s",
                                    num_cores=NC, num_subcores=NS),
        compiler_params=pltpu.CompilerParams(needs_layout_passes=False,
                                             use_tc_tiling_on_sc=False),
        scratch_types=[
            pltpu.VMEM((BPW,), jnp.int32),
            pltpu.VMEM((BPW,), jnp.int32),
            pltpu.VMEM((BPW,), jnp.int32),
            pltpu.VMEM((BPW * NEG,), jnp.int32),
            pltpu.VMEM((BPW, DIM), jnp.float32),
            pltpu.VMEM((BPW, DIM), jnp.float32),
            pltpu.VMEM((BPW, DIM), jnp.float32),
            pltpu.VMEM((BPW, DIM), jnp.float32),
            pltpu.VMEM((BPW,), jnp.float32),
            pltpu.VMEM((NEG_PAD, DIM), jnp.float32),
            pltpu.VMEM((NEG_PAD, DIM), jnp.float32),
            pltpu.VMEM((NEG_PAD,), jnp.float32),
            pltpu.VMEM((NEG_PAD,), jnp.float32),
            pltpu.SemaphoreType.DMA,
            pltpu.SemaphoreType.DMA,
            pltpu.SemaphoreType.DMA,
            pltpu.SemaphoreType.DMA,
            pltpu.SemaphoreType.DMA,
        ],
    )


def _log_sigmoid(x):
    return jnp.minimum(x, 0.0) - jnp.log1p(jnp.exp(-jnp.abs(x)))


def _ls_body(posi_ref, nega_ref, posi1_ref, nega1_ref):
    posi1_ref[...] = _log_sigmoid(posi_ref[...])
    nega1_ref[...] = jnp.mean(_log_sigmoid(-nega_ref[...]), axis=1,
                              keepdims=True)


_ls_call = pl.pallas_call(
    _ls_body,
    out_shape=[
        jax.ShapeDtypeStruct((BATCH, 1), jnp.float32),
        jax.ShapeDtypeStruct((BATCH, 1), jnp.float32),
    ],
)


def kernel(head_part, tail_part, entity_embedding, relation_embedding):
    hidx = head_part[:, 0].astype(jnp.int32)
    ridx = head_part[:, 1].astype(jnp.int32)
    pidx = head_part[:, 2].astype(jnp.int32)
    tidx = tail_part.reshape(-1).astype(jnp.int32)
    posi_score, nega_flat = _sc_score_kernel()(hidx, ridx, pidx, tidx,
                                               entity_embedding,
                                               relation_embedding)
    nega_score = nega_flat.reshape(BATCH, NEG)
    posi1, nega1 = _ls_call(posi_score.reshape(BATCH, 1), nega_score)
    return (posi1.reshape(BATCH), nega1.reshape(BATCH), nega_score)


# speedup vs baseline: 1.0517x; 1.0517x over previous
import functools

import jax
import jax.numpy as jnp
from jax import lax
from jax.experimental import pallas as pl
from jax.experimental.pallas import tpu as pltpu
from jax.experimental.pallas import tpu_sc as plsc

BATCH = 4096
NEG = 200
DIM = 64
NC = 2
NS = 16
NW = NC * NS
BPW = BATCH // NW
LANES = 16
NEG_A = 104
NEG_B = NEG - NEG_A
NEG_PAD = 208


def _score_body(hidx_hbm, ridx_hbm, pidx_hbm, tidx_hbm, ent_hbm, rel_hbm,
                posi_out, nega_out,
                hidx_v, ridx_v, pidx_v, tidx_v,
                hrows, rrows, prows, hr_rows, posi_buf,
                trows0, trows1, sbuf0, sbuf1,
                psem, gsem0, gsem1, ssem0, ssem1):
    wid = lax.axis_index("s") * NC + lax.axis_index("c")
    base = wid * BPW

    pltpu.sync_copy(hidx_hbm.at[pl.ds(base, BPW)], hidx_v)
    pltpu.sync_copy(ridx_hbm.at[pl.ds(base, BPW)], ridx_v)
    pltpu.sync_copy(pidx_hbm.at[pl.ds(base, BPW)], pidx_v)
    pltpu.sync_copy(tidx_hbm.at[pl.ds(base * NEG, BPW * NEG)], tidx_v)

    c_h = pltpu.async_copy(ent_hbm.at[hidx_v], hrows, psem)
    c_r = pltpu.async_copy(rel_hbm.at[ridx_v], rrows, psem)
    c_p = pltpu.async_copy(ent_hbm.at[pidx_v], prows, psem)

    def issue_tail_gather(b, trows, gsem):
        off = b * NEG
        pltpu.async_copy(ent_hbm.at[tidx_v.at[pl.ds(off, NEG_A)]],
                         trows.at[pl.ds(0, NEG_A)], gsem)
        pltpu.async_copy(ent_hbm.at[tidx_v.at[pl.ds(off + NEG_A, NEG_B)]],
                         trows.at[pl.ds(NEG_A, NEG_B)], gsem)

    def drain_tail_gather(trows, gsem):
        pltpu.make_async_copy(ent_hbm.at[pl.ds(0, NEG)],
                              trows.at[pl.ds(0, NEG)], gsem).wait()

    issue_tail_gather(0, trows0, gsem0)
    issue_tail_gather(1, trows1, gsem1)

    c_h.wait()
    c_r.wait()
    c_p.wait()

    lane_iota = lax.iota(jnp.int32, LANES)
    NGRP = NEG_PAD // LANES

    def hr_body(b, carry):
        for c in range(DIM // LANES):
            h = hrows[b, pl.ds(c * LANES, LANES)]
            r = rrows[b, pl.ds(c * LANES, LANES)]
            hr_rows[b, pl.ds(c * LANES, LANES)] = h * r
        return carry

    lax.fori_loop(0, BPW, hr_body, 0, unroll=4)

    def posi_group(g, carry):
        rows = lane_iota + g * LANES

        def d_body(d, acc):
            col = jnp.full((LANES,), d, jnp.int32)
            gh = plsc.load_gather(hr_rows, [rows, col])
            gp = plsc.load_gather(prows, [rows, col])
            return acc + gh * gp

        acc = lax.fori_loop(0, DIM, d_body,
                            jnp.zeros((LANES,), jnp.float32), unroll=8)
        posi_buf[pl.ds(g * LANES, LANES)] = acc
        return carry

    lax.fori_loop(0, BPW // LANES, posi_group, 0, unroll=False)

    def compute_scores(b, trows, sbuf):
        brow = jnp.full((LANES,), b, jnp.int32)

        def d_body(d, accs):
            col = jnp.full((LANES,), d, jnp.int32)
            hv = plsc.load_gather(hr_rows, [brow, col])
            return tuple(
                accs[j] + hv * plsc.load_gather(
                    trows, [lane_iota + j * LANES, col])
                for j in range(NGRP))

        accs = lax.fori_loop(
            0, DIM, d_body,
            tuple(jnp.zeros((LANES,), jnp.float32) for _ in range(NGRP)),
            unroll=4)
        for j in range(NGRP):
            sbuf[pl.ds(j * LANES, LANES)] = accs[j]

    def half(i, b, trows, gsem, sbuf, ssem):
        drain_tail_gather(trows, gsem)

        @pl.when(i > 0)
        def _():
            pltpu.make_async_copy(sbuf.at[pl.ds(0, NEG)],
                                  nega_out.at[pl.ds(0, NEG)], ssem).wait()

        compute_scores(b, trows, sbuf)
        pltpu.async_copy(sbuf.at[pl.ds(0, NEG)],
                         nega_out.at[pl.ds((base + b) * NEG, NEG)], ssem)

        @pl.when(b + 2 < BPW)
        def _():
            issue_tail_gather(b + 2, trows, gsem)

    def outer(i, carry):
        half(i, 2 * i, trows0, gsem0, sbuf0, ssem0)
        half(i, 2 * i + 1, trows1, gsem1, sbuf1, ssem1)
        return carry

    lax.fori_loop(0, BPW // 2, outer, 0, unroll=False)

    pltpu.make_async_copy(sbuf0.at[pl.ds(0, NEG)],
                          nega_out.at[pl.ds(0, NEG)], ssem0).wait()
    pltpu.make_async_copy(sbuf1.at[pl.ds(0, NEG)],
                          nega_out.at[pl.ds(0, NEG)], ssem1).wait()
    pltpu.sync_copy(posi_buf, posi_out.at[pl.ds(base, BPW)])


@functools.cache
def _sc_score_kernel():
    return pl.kernel(
        _score_body,
        out_type=[
            jax.ShapeDtypeStruct((BATCH,), jnp.float32),
            jax.ShapeDtypeStruct((BATCH * NEG,), jnp.float32),
        ],
        mesh=plsc.VectorSubcoreMesh(core_axis_name="c", subcore_axis_name="s",
                                    num_cores=NC, num_subcores=NS),
        compiler_params=pltpu.CompilerParams(needs_layout_passes=False,
                                             use_tc_tiling_on_sc=False),
        scratch_types=[
            pltpu.VMEM((BPW,), jnp.int32),
            pltpu.VMEM((BPW,), jnp.int32),
            pltpu.VMEM((BPW,), jnp.int32),
            pltpu.VMEM((BPW * NEG,), jnp.int32),
            pltpu.VMEM((BPW, DIM), jnp.float32),
            pltpu.VMEM((BPW, DIM), jnp.float32),
            pltpu.VMEM((BPW, DIM), jnp.float32),
            pltpu.VMEM((BPW, DIM), jnp.float32),
            pltpu.VMEM((BPW,), jnp.float32),
            pltpu.VMEM((NEG_PAD, DIM), jnp.float32),
            pltpu.VMEM((NEG_PAD, DIM), jnp.float32),
            pltpu.VMEM((NEG_PAD,), jnp.float32),
            pltpu.VMEM((NEG_PAD,), jnp.float32),
            pltpu.SemaphoreType.DMA,
            pltpu.SemaphoreType.DMA,
            pltpu.SemaphoreType.DMA,
            pltpu.SemaphoreType.DMA,
            pltpu.SemaphoreType.DMA,
        ],
    )


def _log_sigmoid(x):
    return jnp.minimum(x, 0.0) - jnp.log1p(jnp.exp(-jnp.abs(x)))


def _ls_body(posi_ref, nega_ref, posi1_ref, nega1_ref):
    posi1_ref[...] = _log_sigmoid(posi_ref[...])
    nega1_ref[...] = jnp.mean(_log_sigmoid(-nega_ref[...]), axis=1,
                              keepdims=True)


_ls_call = pl.pallas_call(
    _ls_body,
    out_shape=[
        jax.ShapeDtypeStruct((BATCH, 1), jnp.float32),
        jax.ShapeDtypeStruct((BATCH, 1), jnp.float32),
    ],
)


def kernel(head_part, tail_part, entity_embedding, relation_embedding):
    hidx = head_part[:, 0].astype(jnp.int32)
    ridx = head_part[:, 1].astype(jnp.int32)
    pidx = head_part[:, 2].astype(jnp.int32)
    tidx = tail_part.reshape(-1).astype(jnp.int32)
    posi_score, nega_flat = _sc_score_kernel()(hidx, ridx, pidx, tidx,
                                               entity_embedding,
                                               relation_embedding)
    nega_score = nega_flat.reshape(BATCH, NEG)
    posi1, nega1 = _ls_call(posi_score.reshape(BATCH, 1), nega_score)
    return (posi1.reshape(BATCH), nega1.reshape(BATCH), nega_score)


# speedup vs baseline: 1.6391x; 1.5585x over previous
import functools

import jax
import jax.numpy as jnp
from jax import lax
from jax.experimental import pallas as pl
from jax.experimental.pallas import tpu as pltpu
from jax.experimental.pallas import tpu_sc as plsc

BATCH = 4096
NEG = 200
DIM = 64
NC = 2
NS = 16
NW = NC * NS
BPW = BATCH // NW
LANES = 16
NEG_A = 104
NEG_B = NEG - NEG_A
NEG_PAD = 208


def _score_body(hidx_hbm, ridx_hbm, pidx_hbm, tidx_hbm, ent_hbm, rel_hbm,
                posi_out, nega_out,
                hidx_v, ridx_v, pidx_v, tidx_v,
                hrows, rrows, prows, hr_rows, posi_buf,
                trows0, trows1, sbuf0, sbuf1,
                psem, gsem0, gsem1, ssem0, ssem1):
    wid = lax.axis_index("s") * NC + lax.axis_index("c")
    base = wid * BPW

    pltpu.sync_copy(hidx_hbm.at[pl.ds(base, BPW)], hidx_v)
    pltpu.sync_copy(ridx_hbm.at[pl.ds(base, BPW)], ridx_v)
    pltpu.sync_copy(pidx_hbm.at[pl.ds(base, BPW)], pidx_v)
    pltpu.sync_copy(tidx_hbm.at[pl.ds(base * NEG, BPW * NEG)], tidx_v)

    c_h = pltpu.async_copy(ent_hbm.at[hidx_v], hrows, psem)
    c_r = pltpu.async_copy(rel_hbm.at[ridx_v], rrows, psem)
    c_p = pltpu.async_copy(ent_hbm.at[pidx_v], prows, psem)

    def issue_tail_gather(b, trows, gsem):
        off = b * NEG
        pltpu.async_copy(ent_hbm.at[tidx_v.at[pl.ds(off, NEG_A)]],
                         trows.at[pl.ds(0, NEG_A)], gsem)
        pltpu.async_copy(ent_hbm.at[tidx_v.at[pl.ds(off + NEG_A, NEG_B)]],
                         trows.at[pl.ds(NEG_A, NEG_B)], gsem)

    def drain_tail_gather(trows, gsem):
        pltpu.make_async_copy(ent_hbm.at[pl.ds(0, NEG)],
                              trows.at[pl.ds(0, NEG)], gsem).wait()

    issue_tail_gather(0, trows0, gsem0)
    issue_tail_gather(1, trows1, gsem1)

    c_h.wait()
    c_r.wait()
    c_p.wait()

    lane_iota = lax.iota(jnp.int32, LANES)
    NGRP = NEG_PAD // LANES

    def hr_body(b, carry):
        for c in range(DIM // LANES):
            h = hrows[b, pl.ds(c * LANES, LANES)]
            r = rrows[b, pl.ds(c * LANES, LANES)]
            hr_rows[b, pl.ds(c * LANES, LANES)] = h * r
        return carry

    lax.fori_loop(0, BPW, hr_body, 0, unroll=4)

    def posi_group(g, carry):
        rows = lane_iota + g * LANES

        def d_body(d, acc):
            col = (lane_iota + d) & (DIM - 1)
            gh = plsc.load_gather(hr_rows, [rows, col])
            gp = plsc.load_gather(prows, [rows, col])
            return acc + gh * gp

        acc = lax.fori_loop(0, DIM, d_body,
                            jnp.zeros((LANES,), jnp.float32), unroll=8)
        posi_buf[pl.ds(g * LANES, LANES)] = acc
        return carry

    lax.fori_loop(0, BPW // LANES, posi_group, 0, unroll=False)

    def compute_scores(b, trows, sbuf):
        brow = jnp.full((LANES,), b, jnp.int32)

        def d_body(d, accs):
            col = (lane_iota + d) & (DIM - 1)
            hv = plsc.load_gather(hr_rows, [brow, col])
            return tuple(
                accs[j] + hv * plsc.load_gather(
                    trows, [lane_iota + j * LANES, col])
                for j in range(NGRP))

        accs = lax.fori_loop(
            0, DIM, d_body,
            tuple(jnp.zeros((LANES,), jnp.float32) for _ in range(NGRP)),
            unroll=4)
        for j in range(NGRP):
            sbuf[pl.ds(j * LANES, LANES)] = accs[j]

    def half(i, b, trows, gsem, sbuf, ssem):
        drain_tail_gather(trows, gsem)

        @pl.when(i > 0)
        def _():
            pltpu.make_async_copy(sbuf.at[pl.ds(0, NEG)],
                                  nega_out.at[pl.ds(0, NEG)], ssem).wait()

        compute_scores(b, trows, sbuf)
        pltpu.async_copy(sbuf.at[pl.ds(0, NEG)],
                         nega_out.at[pl.ds((base + b) * NEG, NEG)], ssem)

        @pl.when(b + 2 < BPW)
        def _():
            issue_tail_gather(b + 2, trows, gsem)

    def outer(i, carry):
        half(i, 2 * i, trows0, gsem0, sbuf0, ssem0)
        half(i, 2 * i + 1, trows1, gsem1, sbuf1, ssem1)
        return carry

    lax.fori_loop(0, BPW // 2, outer, 0, unroll=False)

    pltpu.make_async_copy(sbuf0.at[pl.ds(0, NEG)],
                          nega_out.at[pl.ds(0, NEG)], ssem0).wait()
    pltpu.make_async_copy(sbuf1.at[pl.ds(0, NEG)],
                          nega_out.at[pl.ds(0, NEG)], ssem1).wait()
    pltpu.sync_copy(posi_buf, posi_out.at[pl.ds(base, BPW)])


@functools.cache
def _sc_score_kernel():
    return pl.kernel(
        _score_body,
        out_type=[
            jax.ShapeDtypeStruct((BATCH,), jnp.float32),
            jax.ShapeDtypeStruct((BATCH * NEG,), jnp.float32),
        ],
        mesh=plsc.VectorSubcoreMesh(core_axis_name="c", subcore_axis_name="s",
                                    num_cores=NC, num_subcores=NS),
        compiler_params=pltpu.CompilerParams(needs_layout_passes=False,
                                             use_tc_tiling_on_sc=False),
        scratch_types=[
            pltpu.VMEM((BPW,), jnp.int32),
            pltpu.VMEM((BPW,), jnp.int32),
            pltpu.VMEM((BPW,), jnp.int32),
            pltpu.VMEM((BPW * NEG,), jnp.int32),
            pltpu.VMEM((BPW, DIM), jnp.float32),
            pltpu.VMEM((BPW, DIM), jnp.float32),
            pltpu.VMEM((BPW, DIM), jnp.float32),
            pltpu.VMEM((BPW, DIM), jnp.float32),
            pltpu.VMEM((BPW,), jnp.float32),
            pltpu.VMEM((NEG_PAD, DIM), jnp.float32),
            pltpu.VMEM((NEG_PAD, DIM), jnp.float32),
            pltpu.VMEM((NEG_PAD,), jnp.float32),
            pltpu.VMEM((NEG_PAD,), jnp.float32),
            pltpu.SemaphoreType.DMA,
            pltpu.SemaphoreType.DMA,
            pltpu.SemaphoreType.DMA,
            pltpu.SemaphoreType.DMA,
            pltpu.SemaphoreType.DMA,
        ],
    )


def _log_sigmoid(x):
    return jnp.minimum(x, 0.0) - jnp.log1p(jnp.exp(-jnp.abs(x)))


def _ls_body(posi_ref, nega_ref, posi1_ref, nega1_ref):
    posi1_ref[...] = _log_sigmoid(posi_ref[...])
    nega1_ref[...] = jnp.mean(_log_sigmoid(-nega_ref[...]), axis=1,
                              keepdims=True)


_ls_call = pl.pallas_call(
    _ls_body,
    out_shape=[
        jax.ShapeDtypeStruct((BATCH, 1), jnp.float32),
        jax.ShapeDtypeStruct((BATCH, 1), jnp.float32),
    ],
)


def kernel(head_part, tail_part, entity_embedding, relation_embedding):
    hidx = head_part[:, 0].astype(jnp.int32)
    ridx = head_part[:, 1].astype(jnp.int32)
    pidx = head_part[:, 2].astype(jnp.int32)
    tidx = tail_part.reshape(-1).astype(jnp.int32)
    posi_score, nega_flat = _sc_score_kernel()(hidx, ridx, pidx, tidx,
                                               entity_embedding,
                                               relation_embedding)
    nega_score = nega_flat.reshape(BATCH, NEG)
    posi1, nega1 = _ls_call(posi_score.reshape(BATCH, 1), nega_score)
    return (posi1.reshape(BATCH), nega1.reshape(BATCH), nega_score)


# speedup vs baseline: 1.6514x; 1.0075x over previous
import functools

import jax
import jax.numpy as jnp
from jax import lax
from jax.experimental import pallas as pl
from jax.experimental.pallas import tpu as pltpu
from jax.experimental.pallas import tpu_sc as plsc

BATCH = 4096
NEG = 200
DIM = 64
NC = 2
NS = 16
NW = NC * NS
BPW = BATCH // NW
LANES = 16
T1_OFF = 56
NEG_B = NEG - 128
NEG_PAD = 208


def _score_body(hidx_hbm, ridx_hbm, pidx_hbm, t0_hbm, t1_hbm, ent_hbm, rel_hbm,
                posi_out, o0_out, o1_out,
                hidx_v, ridx_v, pidx_v, t0_v, t1_v,
                hrows, rrows, prows, hr_rows, posi_buf,
                trows0, trows1, sbuf0, sbuf1,
                psem, gsem0, gsem1, ssem0, ssem1):
    wid = lax.axis_index("s") * NC + lax.axis_index("c")
    base = wid * BPW

    pltpu.sync_copy(hidx_hbm.at[pl.ds(base, BPW)], hidx_v)
    pltpu.sync_copy(ridx_hbm.at[pl.ds(base, BPW)], ridx_v)
    pltpu.sync_copy(pidx_hbm.at[pl.ds(base, BPW)], pidx_v)
    pltpu.sync_copy(t0_hbm.at[pl.ds(base, BPW)], t0_v)
    pltpu.sync_copy(t1_hbm.at[pl.ds(base, BPW)], t1_v)

    c_h = pltpu.async_copy(ent_hbm.at[hidx_v], hrows, psem)
    c_r = pltpu.async_copy(rel_hbm.at[ridx_v], rrows, psem)
    c_p = pltpu.async_copy(ent_hbm.at[pidx_v], prows, psem)

    def issue_tail_gather(b, trows, gsem):
        pltpu.async_copy(ent_hbm.at[t0_v.at[b]],
                         trows.at[pl.ds(0, 128)], gsem)
        pltpu.async_copy(ent_hbm.at[t1_v.at[b, pl.ds(T1_OFF, NEG_B)]],
                         trows.at[pl.ds(128, NEG_B)], gsem)

    def drain_tail_gather(trows, gsem):
        pltpu.make_async_copy(ent_hbm.at[pl.ds(0, NEG)],
                              trows.at[pl.ds(0, NEG)], gsem).wait()

    issue_tail_gather(0, trows0, gsem0)
    issue_tail_gather(1, trows1, gsem1)

    c_h.wait()
    c_r.wait()
    c_p.wait()

    lane_iota = lax.iota(jnp.int32, LANES)
    NGRP = NEG_PAD // LANES

    def hr_body(b, carry):
        for c in range(DIM // LANES):
            h = hrows[b, pl.ds(c * LANES, LANES)]
            r = rrows[b, pl.ds(c * LANES, LANES)]
            hr_rows[b, pl.ds(c * LANES, LANES)] = h * r
        return carry

    lax.fori_loop(0, BPW, hr_body, 0, unroll=4)

    def posi_group(g, carry):
        rows = lane_iota + g * LANES

        def d_body(d, acc):
            col = (lane_iota + d) & (DIM - 1)
            gh = plsc.load_gather(hr_rows, [rows, col])
            gp = plsc.load_gather(prows, [rows, col])
            return acc + gh * gp

        acc = lax.fori_loop(0, DIM, d_body,
                            jnp.zeros((LANES,), jnp.float32), unroll=8)
        posi_buf[pl.ds(g * LANES, LANES)] = acc
        return carry

    lax.fori_loop(0, BPW // LANES, posi_group, 0, unroll=False)

    def compute_scores(b, trows, sbuf):
        brow = jnp.full((LANES,), b, jnp.int32)

        def d_body(d, accs):
            col = (lane_iota + d) & (DIM - 1)
            hv = plsc.load_gather(hr_rows, [brow, col])
            return tuple(
                accs[j] + hv * plsc.load_gather(
                    trows, [lane_iota + j * LANES, col])
                for j in range(NGRP))

        accs = lax.fori_loop(
            0, DIM, d_body,
            tuple(jnp.zeros((LANES,), jnp.float32) for _ in range(NGRP)),
            unroll=4)
        for j in range(NGRP):
            sbuf[pl.ds(j * LANES, LANES)] = accs[j]

    def half(i, b, trows, gsem, sbuf, ssem):
        drain_tail_gather(trows, gsem)

        @pl.when(i > 0)
        def _():
            pltpu.make_async_copy(sbuf.at[pl.ds(0, 128)],
                                  o0_out.at[0], ssem).wait()
            pltpu.make_async_copy(sbuf.at[pl.ds(0, 128)],
                                  o1_out.at[0], ssem).wait()

        compute_scores(b, trows, sbuf)
        pltpu.async_copy(sbuf.at[pl.ds(0, 128)],
                         o0_out.at[base + b], ssem)
        pltpu.async_copy(sbuf.at[pl.ds(72, 128)],
                         o1_out.at[base + b], ssem)

        @pl.when(b + 2 < BPW)
        def _():
            issue_tail_gather(b + 2, trows, gsem)

    def outer(i, carry):
        half(i, 2 * i, trows0, gsem0, sbuf0, ssem0)
        half(i, 2 * i + 1, trows1, gsem1, sbuf1, ssem1)
        return carry

    lax.fori_loop(0, BPW // 2, outer, 0, unroll=False)

    for sbuf, ssem in ((sbuf0, ssem0), (sbuf1, ssem1)):
        pltpu.make_async_copy(sbuf.at[pl.ds(0, 128)],
                              o0_out.at[0], ssem).wait()
        pltpu.make_async_copy(sbuf.at[pl.ds(0, 128)],
                              o1_out.at[0], ssem).wait()
    pltpu.sync_copy(posi_buf, posi_out.at[pl.ds(base, BPW)])


@functools.cache
def _sc_score_kernel():
    return pl.kernel(
        _score_body,
        out_type=[
            jax.ShapeDtypeStruct((BATCH,), jnp.float32),
            jax.ShapeDtypeStruct((BATCH, 128), jnp.float32),
            jax.ShapeDtypeStruct((BATCH, 128), jnp.float32),
        ],
        mesh=plsc.VectorSubcoreMesh(core_axis_name="c", subcore_axis_name="s",
                                    num_cores=NC, num_subcores=NS),
        compiler_params=pltpu.CompilerParams(needs_layout_passes=False,
                                             use_tc_tiling_on_sc=False),
        scratch_types=[
            pltpu.VMEM((BPW,), jnp.int32),
            pltpu.VMEM((BPW,), jnp.int32),
            pltpu.VMEM((BPW,), jnp.int32),
            pltpu.VMEM((BPW, 128), jnp.int32),
            pltpu.VMEM((BPW, 128), jnp.int32),
            pltpu.VMEM((BPW, DIM), jnp.float32),
            pltpu.VMEM((BPW, DIM), jnp.float32),
            pltpu.VMEM((BPW, DIM), jnp.float32),
            pltpu.VMEM((BPW, DIM), jnp.float32),
            pltpu.VMEM((BPW,), jnp.float32),
            pltpu.VMEM((NEG_PAD, DIM), jnp.float32),
            pltpu.VMEM((NEG_PAD, DIM), jnp.float32),
            pltpu.VMEM((NEG_PAD,), jnp.float32),
            pltpu.VMEM((NEG_PAD,), jnp.float32),
            pltpu.SemaphoreType.DMA,
            pltpu.SemaphoreType.DMA,
            pltpu.SemaphoreType.DMA,
            pltpu.SemaphoreType.DMA,
            pltpu.SemaphoreType.DMA,
        ],
    )


def _log_sigmoid(x):
    return jnp.minimum(x, 0.0) - jnp.log1p(jnp.exp(-jnp.abs(x)))


def _ls_body(posi_ref, o0_ref, o1_ref, posi1_ref, nega1_ref, nega_ref):
    posi1_ref[...] = _log_sigmoid(posi_ref[...])
    n0 = o0_ref[...]
    n1 = o1_ref[...][:, T1_OFF:]
    nega_ref[:, :128] = n0
    nega_ref[:, 128:] = n1
    s = (jnp.sum(_log_sigmoid(-n0), axis=1, keepdims=True)
         + jnp.sum(_log_sigmoid(-n1), axis=1, keepdims=True))
    nega1_ref[...] = s * (1.0 / NEG)


_ls_call = pl.pallas_call(
    _ls_body,
    out_shape=[
        jax.ShapeDtypeStruct((BATCH // 128, 128), jnp.float32),
        jax.ShapeDtypeStruct((BATCH, 1), jnp.float32),
        jax.ShapeDtypeStruct((BATCH, NEG), jnp.float32),
    ],
)


def kernel(head_part, tail_part, entity_embedding, relation_embedding):
    hidx = head_part[:, 0].astype(jnp.int32)
    ridx = head_part[:, 1].astype(jnp.int32)
    pidx = head_part[:, 2].astype(jnp.int32)
    t0 = tail_part[:, :128].astype(jnp.int32)
    t1 = tail_part[:, NEG - 128:].astype(jnp.int32)
    posi_score, o0, o1 = _sc_score_kernel()(hidx, ridx, pidx, t0, t1,
                                            entity_embedding,
                                            relation_embedding)
    posi1, nega1, nega_score = _ls_call(
        posi_score.reshape(BATCH // 128, 128), o0, o1)
    return (posi1.reshape(BATCH), nega1.reshape(BATCH), nega_score)


# speedup vs baseline: 2.0879x; 1.2644x over previous
import functools

import jax
import jax.numpy as jnp
from jax import lax
from jax.experimental import pallas as pl
from jax.experimental.pallas import tpu as pltpu
from jax.experimental.pallas import tpu_sc as plsc

BATCH = 4096
NEG = 200
DIM = 64
NC = 2
NS = 16
NW = NC * NS
BPW = BATCH // NW
LANES = 16
T1_OFF = 56
NEG_B = NEG - 128
NEG_PAD = 208


def _score_body(hidx_hbm, pidx_hbm, t0_hbm, t1_hbm, ent_hbm, rel2_hbm,
                posi_out, o0_out, o1_out,
                hidx_v, pidx_v, t0_v, t1_v, rel2_v,
                hrows, prows, hr_rows, posi_buf,
                trows0, trows1, sbuf0, sbuf1,
                psem, gsem0, gsem1, ssem0, ssem1):
    wid = lax.axis_index("s") * NC + lax.axis_index("c")
    base = wid * BPW

    pltpu.sync_copy(hidx_hbm.at[pl.ds(base, BPW)], hidx_v)
    pltpu.sync_copy(pidx_hbm.at[pl.ds(base, BPW)], pidx_v)
    pltpu.sync_copy(t0_hbm.at[pl.ds(base, BPW)], t0_v)
    pltpu.sync_copy(t1_hbm.at[pl.ds(base, BPW)], t1_v)
    pltpu.sync_copy(rel2_hbm.at[pl.ds(base // 2, BPW // 2)], rel2_v)

    c_h = pltpu.async_copy(ent_hbm.at[hidx_v], hrows, psem)
    c_p = pltpu.async_copy(ent_hbm.at[pidx_v], prows, psem)

    def issue_tail_gather(b, trows, gsem):
        pltpu.async_copy(ent_hbm.at[t0_v.at[b]],
                         trows.at[pl.ds(0, 128)], gsem)
        pltpu.async_copy(ent_hbm.at[t1_v.at[b, pl.ds(T1_OFF, NEG_B)]],
                         trows.at[pl.ds(128, NEG_B)], gsem)

    def drain_tail_gather(trows, gsem):
        pltpu.make_async_copy(ent_hbm.at[pl.ds(0, NEG)],
                              trows.at[pl.ds(0, NEG)], gsem).wait()

    issue_tail_gather(0, trows0, gsem0)
    issue_tail_gather(1, trows1, gsem1)

    c_h.wait()
    c_p.wait()

    lane_iota = lax.iota(jnp.int32, LANES)
    NGRP = NEG_PAD // LANES

    def hr_body(bp, carry):
        for half in range(2):
            b = 2 * bp + half
            for c in range(DIM // LANES):
                h = hrows[b, pl.ds(c * LANES, LANES)]
                r = rel2_v[bp, pl.ds(half * DIM + c * LANES, LANES)]
                hr_rows[b, pl.ds(c * LANES, LANES)] = h * r
        return carry

    lax.fori_loop(0, BPW // 2, hr_body, 0, unroll=4)

    def posi_group(g, carry):
        rows = lane_iota + g * LANES

        def d_body(d, acc):
            col = (lane_iota + d) & (DIM - 1)
            gh = plsc.load_gather(hr_rows, [rows, col])
            gp = plsc.load_gather(prows, [rows, col])
            return acc + gh * gp

        acc = lax.fori_loop(0, DIM, d_body,
                            jnp.zeros((LANES,), jnp.float32), unroll=8)
        posi_buf[pl.ds(g * LANES, LANES)] = acc
        return carry

    lax.fori_loop(0, BPW // LANES, posi_group, 0, unroll=False)

    def compute_scores(b, trows, sbuf):
        brow = jnp.full((LANES,), b, jnp.int32)

        def d_body(d, accs):
            col = (lane_iota + d) & (DIM - 1)
            hv = plsc.load_gather(hr_rows, [brow, col])
            return tuple(
                accs[j] + hv * plsc.load_gather(
                    trows, [lane_iota + j * LANES, col])
                for j in range(NGRP))

        accs = lax.fori_loop(
            0, DIM, d_body,
            tuple(jnp.zeros((LANES,), jnp.float32) for _ in range(NGRP)),
            unroll=4)
        for j in range(NGRP):
            sbuf[pl.ds(j * LANES, LANES)] = accs[j]

    def half(i, b, trows, gsem, sbuf, ssem):
        drain_tail_gather(trows, gsem)

        @pl.when(i > 0)
        def _():
            pltpu.make_async_copy(sbuf.at[pl.ds(0, 128)],
                                  o0_out.at[0], ssem).wait()
            pltpu.make_async_copy(sbuf.at[pl.ds(0, 128)],
                                  o1_out.at[0], ssem).wait()

        compute_scores(b, trows, sbuf)
        pltpu.async_copy(sbuf.at[pl.ds(0, 128)],
                         o0_out.at[base + b], ssem)
        pltpu.async_copy(sbuf.at[pl.ds(72, 128)],
                         o1_out.at[base + b], ssem)

        @pl.when(b + 2 < BPW)
        def _():
            issue_tail_gather(b + 2, trows, gsem)

    def outer(i, carry):
        half(i, 2 * i, trows0, gsem0, sbuf0, ssem0)
        half(i, 2 * i + 1, trows1, gsem1, sbuf1, ssem1)
        return carry

    lax.fori_loop(0, BPW // 2, outer, 0, unroll=False)

    for sbuf, ssem in ((sbuf0, ssem0), (sbuf1, ssem1)):
        pltpu.make_async_copy(sbuf.at[pl.ds(0, 128)],
                              o0_out.at[0], ssem).wait()
        pltpu.make_async_copy(sbuf.at[pl.ds(0, 128)],
                              o1_out.at[0], ssem).wait()
    pltpu.sync_copy(posi_buf, posi_out.at[pl.ds(base, BPW)])


@functools.cache
def _sc_score_kernel():
    return pl.kernel(
        _score_body,
        out_type=[
            jax.ShapeDtypeStruct((BATCH,), jnp.float32),
            jax.ShapeDtypeStruct((BATCH, 128), jnp.float32),
            jax.ShapeDtypeStruct((BATCH, 128), jnp.float32),
        ],
        mesh=plsc.VectorSubcoreMesh(core_axis_name="c", subcore_axis_name="s",
                                    num_cores=NC, num_subcores=NS),
        compiler_params=pltpu.CompilerParams(needs_layout_passes=False,
                                             use_tc_tiling_on_sc=False),
        scratch_types=[
            pltpu.VMEM((BPW,), jnp.int32),
            pltpu.VMEM((BPW,), jnp.int32),
            pltpu.VMEM((BPW, 128), jnp.int32),
            pltpu.VMEM((BPW, 128), jnp.int32),
            pltpu.VMEM((BPW // 2, 128), jnp.float32),
            pltpu.VMEM((BPW, DIM), jnp.float32),
            pltpu.VMEM((BPW, DIM), jnp.float32),
            pltpu.VMEM((BPW, DIM), jnp.float32),
            pltpu.VMEM((BPW,), jnp.float32),
            pltpu.VMEM((NEG_PAD, DIM), jnp.float32),
            pltpu.VMEM((NEG_PAD, DIM), jnp.float32),
            pltpu.VMEM((NEG_PAD,), jnp.float32),
            pltpu.VMEM((NEG_PAD,), jnp.float32),
            pltpu.SemaphoreType.DMA,
            pltpu.SemaphoreType.DMA,
            pltpu.SemaphoreType.DMA,
            pltpu.SemaphoreType.DMA,
            pltpu.SemaphoreType.DMA,
        ],
    )


def _pre_body(tp_ref, t0_ref, t1_ref):
    t0_ref[...] = tp_ref[:, :128]
    t1_ref[...] = tp_ref[:, NEG - 128:]


_pre_call = pl.pallas_call(
    _pre_body,
    out_shape=[
        jax.ShapeDtypeStruct((BATCH, 128), jnp.int32),
        jax.ShapeDtypeStruct((BATCH, 128), jnp.int32),
    ],
)


def _log_sigmoid(x):
    return jnp.minimum(x, 0.0) - jnp.log1p(jnp.exp(-jnp.abs(x)))


def _ls_body(posi_ref, o0_ref, o1_ref, posi1_ref, nega1_ref, nega_ref):
    posi1_ref[...] = _log_sigmoid(posi_ref[...])
    n0 = o0_ref[...]
    n1 = o1_ref[...][:, T1_OFF:]
    nega_ref[:, :128] = n0
    nega_ref[:, 128:] = n1
    s = (jnp.sum(_log_sigmoid(-n0), axis=1, keepdims=True)
         + jnp.sum(_log_sigmoid(-n1), axis=1, keepdims=True))
    nega1_ref[...] = s * (1.0 / NEG)


_ls_call = pl.pallas_call(
    _ls_body,
    out_shape=[
        jax.ShapeDtypeStruct((BATCH // 128, 128), jnp.float32),
        jax.ShapeDtypeStruct((BATCH, 1), jnp.float32),
        jax.ShapeDtypeStruct((BATCH, NEG), jnp.float32),
    ],
)


def kernel(head_part, tail_part, entity_embedding, relation_embedding):
    hidx = head_part[:, 0].astype(jnp.int32)
    ridx = head_part[:, 1].astype(jnp.int32)
    pidx = head_part[:, 2].astype(jnp.int32)
    t0, t1 = _pre_call(tail_part.astype(jnp.int32))
    rel2 = jnp.take(relation_embedding, ridx, axis=0).reshape(
        BATCH // 2, 2 * DIM)
    posi_score, o0, o1 = _sc_score_kernel()(hidx, pidx, t0, t1,
                                            entity_embedding, rel2)
    posi1, nega1, nega_score = _ls_call(
        posi_score.reshape(BATCH // 128, 128), o0, o1)
    return (posi1.reshape(BATCH), nega1.reshape(BATCH), nega_score)
